# baseline (speedup 1.0000x reference)
import functools

import jax
import jax.numpy as jnp
from jax import lax
from jax.experimental import pallas as pl
from jax.experimental.pallas import tpu as pltpu
from jax.experimental.pallas import tpu_sc as plsc

HIDDEN = 768
EPS = 1e-5
BATCH = 64
SEQ = 512
L = 16
NW = 32
PPW = SEQ // NW
NCHUNK = HIDDEN // L
BPC = 2
ROWS = BPC * L
NDMA = BATCH // BPC


def _rsqrt_newton_scalar(v):
    i = lax.bitcast_convert_type(v, jnp.int32)
    i = 0x5F3759DF - lax.shift_right_logical(i, 1)
    y = lax.bitcast_convert_type(i, jnp.float32)
    for _ in range(4):
        y = y * (1.5 - 0.5 * v * y * y)
    return y


def _sc_fused(ids, tt, word_emb, pos_emb, type_emb, gamma, beta):
    mesh = plsc.VectorSubcoreMesh(core_axis_name="c", subcore_axis_name="s")

    @functools.partial(
        pl.kernel, mesh=mesh,
        out_type=jax.ShapeDtypeStruct((BATCH, SEQ, HIDDEN), jnp.float32),
        compiler_params=pltpu.CompilerParams(
            use_tc_tiling_on_sc=False, needs_layout_passes=False),
        scratch_types=[
            pltpu.VMEM((L * BATCH,), jnp.int32),
            pltpu.VMEM((L * BATCH,), jnp.int32),
            pltpu.VMEM((2, HIDDEN), jnp.float32),
            pltpu.VMEM((2 * L, HIDDEN), jnp.float32),
            pltpu.VMEM((HIDDEN,), jnp.float32),
            pltpu.VMEM((HIDDEN,), jnp.float32),
            pltpu.VMEM((ROWS, HIDDEN), jnp.float32),
            pltpu.VMEM((ROWS, HIDDEN), jnp.float32),
            pltpu.VMEM((BPC, L, HIDDEN), jnp.float32),
            pltpu.SemaphoreType.DMA,
            pltpu.SemaphoreType.DMA,
            pltpu.SemaphoreType.DMA,
        ],
    )
    def k(ids_hbm, tt_hbm, word_hbm, pos_hbm, type_hbm, gamma_hbm, beta_hbm,
          out_hbm, idx_v, ttv, type_v, combo_v, g_v, b_v,
          ws0, ws1, os, sem_g0, sem_g1, sem_o):
        wid = lax.axis_index("s") * 2 + lax.axis_index("c")
        p0 = pl.multiple_of(wid * PPW, PPW)
        tok0 = pl.multiple_of(wid * (L * BATCH), L * BATCH)

        pltpu.sync_copy(ids_hbm.at[pl.ds(tok0, L * BATCH)], idx_v)
        pltpu.sync_copy(tt_hbm.at[pl.ds(tok0, L * BATCH)], ttv)
        pltpu.sync_copy(type_hbm.at[pl.ds(0, 2)], type_v)
        pltpu.sync_copy(gamma_hbm, g_v)
        pltpu.sync_copy(beta_hbm, b_v)
        pltpu.sync_copy(pos_hbm.at[pl.ds(p0, L)], combo_v.at[pl.ds(0, L)])
        pltpu.sync_copy(pos_hbm.at[pl.ds(p0, L)], combo_v.at[pl.ds(L, L)])
        for r in range(L):
            def build_chunk(c, _, r=r):
                sl = pl.ds(c * L, L)
                combo_v[r, sl] = combo_v[r, sl] + type_v[0, sl]
                combo_v[L + r, sl] = combo_v[L + r, sl] + type_v[1, sl]
                return 0
            lax.fori_loop(0, NCHUNK, build_chunk, 0, unroll=8)

        iota = lax.iota(jnp.int32, L)
        inv_h = jnp.float32(1.0 / HIDDEN)
        bufs = ((ws0, sem_g0), (ws1, sem_g1))

        def id_slice(ch):
            return idx_v.at[pl.ds(ch * ROWS, ROWS)]

        pltpu.async_copy(word_hbm.at[id_slice(0)], ws0, sem_g0)

        def group(g, _):
            for j in range(2):
                ch = 2 * g + j
                ws, sem_g = bufs[j]
                ws_n, sem_g_n = bufs[1 - j]
                pltpu.make_async_copy(word_hbm.at[id_slice(ch)], ws, sem_g).wait()
                chn = jnp.minimum(ch + 1, NDMA - 1)
                pltpu.async_copy(word_hbm.at[id_slice(chn)], ws_n, sem_g_n)

                stats = []
                for half in range(BPC):
                    r0 = half * L
                    cvec = ttv[pl.ds(ch * ROWS + r0, L)] * L + iota
                    ci = [cvec[i] for i in range(L)]

                    def pass1(c, carry, ws=ws, ci=ci, r0=r0):
                        sl = pl.ds(c * L, L)
                        out = []
                        for i in range(L):
                            x = ws[r0 + i, sl] + combo_v[ci[i], sl]
                            ws[r0 + i, sl] = x
                            out.append(carry[i] + x)
                            out.append(carry[L + i] + x * x)
                        return tuple(out[::2]) + tuple(out[1::2])

                    zero = jnp.zeros((L,), jnp.float32)
                    carry = lax.fori_loop(0, NCHUNK, pass1, (zero,) * (2 * L),
                                          unroll=4)

                    ms, rs = [], []
                    for i in range(L):
                        s1 = jnp.sum(carry[i])
                        s2 = jnp.sum(carry[L + i])
                        m = s1 * inv_h
                        ms.append(m)
                        rs.append(
                            _rsqrt_newton_scalar(s2 * inv_h - m * m + EPS))
                    stats.append((ms, rs))

                @pl.when(ch >= 1)
                def _():
                    pltpu.make_async_copy(
                        os, out_hbm.at[pl.ds(ch * BPC, BPC), pl.ds(p0, L)],
                        sem_o).wait()

                for half in range(BPC):
                    r0 = half * L
                    ms, rs = stats[half]

                    def pass2(c, _, ms=ms, rs=rs, r0=r0, half=half, ws=ws):
                        sl = pl.ds(c * L, L)
                        gvec = g_v[sl]
                        bvec = b_v[sl]
                        for i in range(L):
                            a = gvec * rs[i]
                            os[half, i, sl] = (ws[r0 + i, sl] - ms[i]) * a + bvec
                        return 0

                    lax.fori_loop(0, NCHUNK, pass2, 0, unroll=4)

                pltpu.async_copy(
                    os, out_hbm.at[pl.ds(ch * BPC, BPC), pl.ds(p0, L)], sem_o)
            return 0

        lax.fori_loop(0, NDMA // 2, group, 0)

        pltpu.make_async_copy(word_hbm.at[id_slice(0)], ws0, sem_g0).wait()
        pltpu.make_async_copy(
            os, out_hbm.at[pl.ds(0, BPC), pl.ds(p0, L)], sem_o).wait()

    return k(ids, tt, word_emb, pos_emb, type_emb, gamma, beta)


def kernel(input_ids, token_type_ids, word_emb, pos_emb, type_emb, gamma, beta):
    def perm(a):
        return (a.astype(jnp.int32).reshape(BATCH, NW, PPW)
                .transpose(1, 0, 2).reshape(-1))
    return _sc_fused(perm(input_ids), perm(token_type_ids),
                     word_emb, pos_emb, type_emb, gamma, beta)


# speedup vs baseline: 4.7078x; 4.7078x over previous
import functools

import jax
import jax.numpy as jnp
from jax import lax
from jax.experimental import pallas as pl
from jax.experimental.pallas import tpu as pltpu
from jax.experimental.pallas import tpu_sc as plsc

HIDDEN = 768
EPS = 1e-5
NUM_WORKERS = 32


def _sc_gather(table, idx):
    _, D = table.shape
    B = idx.shape[0]
    b_per_w = B // NUM_WORKERS
    C = 128
    n_chunks = b_per_w // C
    mesh = plsc.VectorSubcoreMesh(core_axis_name="c", subcore_axis_name="s")

    @functools.partial(
        pl.kernel, mesh=mesh,
        out_type=jax.ShapeDtypeStruct((B, D), jnp.float32),
        scratch_types=[
            pltpu.VMEM((C,), jnp.int32),
            pltpu.VMEM((C, D), jnp.float32),
            pltpu.SemaphoreType.DMA,
        ],
    )
    def k(table_hbm, idx_hbm, out_hbm, idx_v, rows_v, sem):
        wid = lax.axis_index("s") * 2 + lax.axis_index("c")
        base = wid * b_per_w

        def body(i, carry):
            off = base + i * C
            pltpu.sync_copy(idx_hbm.at[pl.ds(off, C)], idx_v)
            pltpu.async_copy(table_hbm.at[idx_v], rows_v, sem).wait()
            pltpu.sync_copy(rows_v, out_hbm.at[pl.ds(off, C)])
            return carry

        lax.fori_loop(0, n_chunks, body, 0)

    return k(table, idx)


def _tc_layernorm(x, pos_emb, tt3, type_emb, gamma2, beta2):
    BATCH, SEQ, _ = x.shape
    BB = 4

    def body(x_ref, pos_ref, tt_ref, type_ref, g_ref, b_ref, o_ref):
        pos = pos_ref[...]
        t0 = type_ref[0]
        t1 = type_ref[1]
        g = g_ref[0]
        bb = b_ref[0]
        for i in range(BB):
            xb = x_ref[i]
            ttc = tt_ref[i]
            e = xb + pos + (t0[None, :] * (1.0 - ttc) + t1[None, :] * ttc)
            mean = jnp.mean(e, axis=-1, keepdims=True)
            c = e - mean
            var = jnp.mean(c * c, axis=-1, keepdims=True)
            o_ref[i] = c * lax.rsqrt(var + EPS) * g[None, :] + bb[None, :]

    return pl.pallas_call(
        body,
        grid=(BATCH // BB,),
        in_specs=[
            pl.BlockSpec((BB, SEQ, HIDDEN), lambda b: (b, 0, 0)),
            pl.BlockSpec((SEQ, HIDDEN), lambda b: (0, 0)),
            pl.BlockSpec((BB, SEQ, 1), lambda b: (b, 0, 0)),
            pl.BlockSpec((2, HIDDEN), lambda b: (0, 0)),
            pl.BlockSpec((1, HIDDEN), lambda b: (0, 0)),
            pl.BlockSpec((1, HIDDEN), lambda b: (0, 0)),
        ],
        out_specs=pl.BlockSpec((BB, SEQ, HIDDEN), lambda b: (b, 0, 0)),
        out_shape=jax.ShapeDtypeStruct((BATCH, SEQ, HIDDEN), jnp.float32),
    )(x, pos_emb, tt3, type_emb, gamma2, beta2)


def kernel(input_ids, token_type_ids, word_emb, pos_emb, type_emb, gamma, beta):
    B, S = input_ids.shape
    ids = input_ids.reshape(-1).astype(jnp.int32)
    gathered = _sc_gather(word_emb, ids)
    x = gathered.reshape(B, S, HIDDEN)
    tt3 = token_type_ids.reshape(B, S, 1).astype(jnp.float32)
    return _tc_layernorm(
        x, pos_emb, tt3, type_emb,
        gamma.reshape(1, HIDDEN), beta.reshape(1, HIDDEN),
    )
